# baseline (speedup 1.0000x reference)
import functools

import jax
import jax.numpy as jnp
from jax import lax
from jax.experimental import pallas as pl
from jax.experimental.pallas import tpu as pltpu
from jax.experimental.pallas import tpu_sc as plsc

N, B, D, LD, C = 8192, 128, 128, 32, 64
BLK = 1024
NB = N // BLK
CD = D * 5
CLD = LD * 4
HI = jax.lax.Precision.HIGHEST


def _chunks7(v):
    parts = [((v >> s) & 127).astype(jnp.bfloat16) for s in (0, 7, 14, 21, 28)]
    return jnp.concatenate(parts, axis=-1)


def _chunks8(v):
    parts = [((v >> s) & 255).astype(jnp.bfloat16) for s in (0, 8, 16, 24)]
    return jnp.concatenate(parts, axis=-1)


def _bits(f):
    return jax.lax.bitcast_convert_type(f, jnp.int32)


def _dot_t(a, b, prec=None):
    return jax.lax.dot_general(a, b, (((1,), (1,)), ((), ())),
                               precision=prec, preferred_element_type=jnp.float32)


def _dot(a, b):
    return jax.lax.dot_general(a, b, (((1,), (0,)), ((), ())),
                               preferred_element_type=jnp.float32)


def _split3(v):
    hi = v.astype(jnp.bfloat16)
    r1 = v - hi.astype(jnp.float32)
    mid = r1.astype(jnp.bfloat16)
    lo = (r1 - mid.astype(jnp.float32)).astype(jnp.bfloat16)
    return hi, mid, lo


def _match_kernel(x_ref, sf_ref, midx_ref, xc_ref):
    j = pl.program_id(0)

    @pl.when(j == 0)
    def _init():
        xc_ref[...] = _chunks7(_bits(x_ref[...]))
        midx_ref[...] = jnp.full_like(midx_ref, N)

    sfc = _chunks7(_bits(sf_ref[...]))
    xc = xc_ref[...]
    g = _dot_t(xc, sfc)
    nx = jnp.sum(xc.astype(jnp.float32) ** 2, axis=1)
    nf = jnp.sum(sfc.astype(jnp.float32) ** 2, axis=1)
    m2 = nx[:, None] + nf[None, :] - 2.0 * g
    il = jax.lax.broadcasted_iota(jnp.int32, (B, BLK), 1)
    lidx = jnp.min(jnp.where(m2 == 0.0, il, BLK), axis=1)
    cand = jnp.where(lidx < BLK, j * BLK + lidx, N)
    midx_ref[0, :] = jnp.minimum(midx_ref[0, :], cand)


def _sc_gather_kernel(idx_hbm, d1_hbm, d2_hbm, o1_hbm, o2_hbm,
                      idx_v, rows_v, sem):
    base = lax.axis_index("s") * (B // 16)
    pltpu.sync_copy(idx_hbm.at[pl.ds(base, B // 16)], idx_v)
    pltpu.async_copy(d1_hbm.at[idx_v], rows_v, sem).wait()
    pltpu.sync_copy(rows_v, o1_hbm.at[pl.ds(base, B // 16)])
    pltpu.async_copy(d2_hbm.at[idx_v], rows_v, sem).wait()
    pltpu.sync_copy(rows_v, o2_hbm.at[pl.ds(base, B // 16)])


def _main_kernel(midx_ref, w1_ref, b1_ref, w2_ref, b2_ref, u1_ref, u2_ref,
                 d1f_ref, d1l_ref, d2f_ref, d2l_ref, slb_ref, out_ref,
                 x1_ref, x2_ref, u1c_ref, u2c_ref, oh1_ref, oh2_ref,
                 nx1_ref, nx2_ref, num1_ref, num2_ref):
    j = pl.program_id(0)
    sides = (
        (x1_ref, w1_ref, b1_ref, u1_ref, u1c_ref, oh1_ref, nx1_ref,
         d1f_ref, d1l_ref, num1_ref),
        (x2_ref, w2_ref, b2_ref, u2_ref, u2c_ref, oh2_ref, nx2_ref,
         d2f_ref, d2l_ref, num2_ref),
    )

    @pl.when(j == 0)
    def _init():
        ohq = (midx_ref[0, :][:, None]
               == jax.lax.broadcasted_iota(jnp.int32, (B, N), 1)
               ).astype(jnp.bfloat16)
        for (x_ref, w_ref, b_ref, u_ref, uc_ref, oh_ref, nx_ref,
             df_ref, _dl, num_ref) in sides:
            hi, mid, lo = _split3(df_ref[...])
            x_ref[...] = _dot(ohq, hi) + _dot(ohq, mid) + _dot(ohq, lo)
            u = u_ref[...]
            uc_ref[...] = _chunks8(_bits(u))
            xg = x_ref[...]
            y = jax.lax.dot_general(xg, w_ref[...], (((1,), (0,)), ((), ())),
                                    precision=HI,
                                    preferred_element_type=jnp.float32)
            y = y + b_ref[0, :][None, :]
            ny = jnp.sum(y * y, axis=1)
            nuf = jnp.sum(u * u, axis=1)
            dq = ny[:, None] + nuf[None, :] - 2.0 * _dot_t(y, u, HI)
            mn = jnp.min(dq, axis=1, keepdims=True)
            cb = jax.lax.broadcasted_iota(jnp.int32, (B, C), 1)
            cidx = jnp.min(jnp.where(dq == mn, cb, C), axis=1)
            oh_ref[...] = (cb == cidx[:, None]).astype(jnp.bfloat16)
            nx_ref[0, :] = jnp.sum(xg * xg, axis=1)
            num_ref[...] = jnp.zeros_like(num_ref)

    slb = slb_ref[...]
    slb_ext = jnp.concatenate(
        [slb, jnp.ones((BLK, 1), jnp.float32)], axis=1).astype(jnp.bfloat16)
    for (x_ref, _w, _b, _u, uc_ref, oh_ref, nx_ref,
         df_ref, dl_ref, num_ref) in sides:
        f = df_ref[pl.ds(pl.multiple_of(j * BLK, BLK), BLK), :]
        uc = uc_ref[...]
        lc = _chunks8(_bits(dl_ref[...]))
        nl = jnp.sum(lc.astype(jnp.float32) ** 2, axis=1)
        nu = jnp.sum(uc.astype(jnp.float32) ** 2, axis=1)
        m2l = nl[:, None] + nu[None, :] - 2.0 * _dot_t(lc, uc)
        e = (m2l == 0.0).astype(jnp.bfloat16)
        mask = _dot_t(oh_ref[...], e)
        g = _dot_t(x_ref[...], f, HI)
        nf = jnp.sum(f * f, axis=1)
        sq = nx_ref[0, :][:, None] + nf[None, :] - 2.0 * g
        expo = (jnp.exp(-sq) * mask).astype(jnp.bfloat16)
        num_ref[...] += jax.lax.dot_general(
            expo, slb_ext, (((1,), (0,)), ((), ())),
            preferred_element_type=jnp.float32)

    @pl.when(j == NB - 1)
    def _fin():
        n1 = num1_ref[...]
        n2 = num2_ref[...]
        out_ref[...] = 0.5 * (n1[:, :LD] / n1[:, LD:LD + 1]
                              + n2[:, :LD] / n2[:, LD:LD + 1])


def kernel(x, star_features, star_labels, d1_features, d1_labels,
           d2_features, d2_labels, unique1, unique2, W1, b1, W2, b2):
    f32 = jnp.float32
    midx = pl.pallas_call(
        _match_kernel,
        grid=(NB,),
        in_specs=[
            pl.BlockSpec((B, D), lambda j: (0, 0)),
            pl.BlockSpec((BLK, D), lambda j: (j, 0)),
        ],
        out_specs=pl.BlockSpec((1, B), lambda j: (0, 0)),
        out_shape=jax.ShapeDtypeStruct((1, B), jnp.int32),
        scratch_shapes=[
            pltpu.VMEM((B, CD), jnp.bfloat16),
        ],
    )(x, star_features)

    s = pl.pallas_call(
        _main_kernel,
        grid=(NB,),
        in_specs=[
            pl.BlockSpec((1, B), lambda j: (0, 0)),
            pl.BlockSpec((D, LD), lambda j: (0, 0)),
            pl.BlockSpec((1, LD), lambda j: (0, 0)),
            pl.BlockSpec((D, LD), lambda j: (0, 0)),
            pl.BlockSpec((1, LD), lambda j: (0, 0)),
            pl.BlockSpec((C, LD), lambda j: (0, 0)),
            pl.BlockSpec((C, LD), lambda j: (0, 0)),
            pl.BlockSpec((N, D), lambda j: (0, 0)),
            pl.BlockSpec((BLK, LD), lambda j: (j, 0)),
            pl.BlockSpec((N, D), lambda j: (0, 0)),
            pl.BlockSpec((BLK, LD), lambda j: (j, 0)),
            pl.BlockSpec((BLK, LD), lambda j: (j, 0)),
        ],
        out_specs=pl.BlockSpec((B, LD), lambda j: (0, 0)),
        out_shape=jax.ShapeDtypeStruct((B, LD), f32),
        scratch_shapes=[
            pltpu.VMEM((B, D), f32),
            pltpu.VMEM((B, D), f32),
            pltpu.VMEM((C, CLD), jnp.bfloat16),
            pltpu.VMEM((C, CLD), jnp.bfloat16),
            pltpu.VMEM((B, C), jnp.bfloat16),
            pltpu.VMEM((B, C), jnp.bfloat16),
            pltpu.VMEM((1, B), f32),
            pltpu.VMEM((1, B), f32),
            pltpu.VMEM((B, LD + 1), f32),
            pltpu.VMEM((B, LD + 1), f32),
        ],
    )(midx, W1, b1.reshape(1, LD), W2, b2.reshape(1, LD),
      unique1, unique2, d1_features, d1_labels, d2_features, d2_labels,
      star_labels)
    return s


# speedup vs baseline: 1.0508x; 1.0508x over previous
import functools

import jax
import jax.numpy as jnp
from jax import lax
from jax.experimental import pallas as pl
from jax.experimental.pallas import tpu as pltpu
from jax.experimental.pallas import tpu_sc as plsc

N, B, D, LD, C = 8192, 128, 128, 32, 64
BLK = 2048
NB = N // BLK
CD = D * 5
CLD = LD * 4
HI = jax.lax.Precision.HIGHEST


def _chunks7(v):
    parts = [((v >> s) & 127).astype(jnp.bfloat16) for s in (0, 7, 14, 21, 28)]
    return jnp.concatenate(parts, axis=-1)


def _chunks8(v):
    parts = [((v >> s) & 255).astype(jnp.bfloat16) for s in (0, 8, 16, 24)]
    return jnp.concatenate(parts, axis=-1)


def _bits(f):
    return jax.lax.bitcast_convert_type(f, jnp.int32)


def _dot_t(a, b, prec=None):
    return jax.lax.dot_general(a, b, (((1,), (1,)), ((), ())),
                               precision=prec, preferred_element_type=jnp.float32)


def _dot(a, b):
    return jax.lax.dot_general(a, b, (((1,), (0,)), ((), ())),
                               preferred_element_type=jnp.float32)


def _split3(v):
    hi = v.astype(jnp.bfloat16)
    r1 = v - hi.astype(jnp.float32)
    mid = r1.astype(jnp.bfloat16)
    lo = (r1 - mid.astype(jnp.float32)).astype(jnp.bfloat16)
    return hi, mid, lo


def _match_kernel(x_ref, sf_ref, midx_ref, xc_ref):
    j = pl.program_id(0)

    @pl.when(j == 0)
    def _init():
        xc_ref[...] = _chunks7(_bits(x_ref[...]))
        midx_ref[...] = jnp.full_like(midx_ref, N)

    sfc = _chunks7(_bits(sf_ref[...]))
    xc = xc_ref[...]
    g = _dot_t(xc, sfc)
    nx = jnp.sum(xc.astype(jnp.float32) ** 2, axis=1)
    nf = jnp.sum(sfc.astype(jnp.float32) ** 2, axis=1)
    m2 = nx[:, None] + nf[None, :] - 2.0 * g
    il = jax.lax.broadcasted_iota(jnp.int32, (B, BLK), 1)
    lidx = jnp.min(jnp.where(m2 == 0.0, il, BLK), axis=1)
    cand = jnp.where(lidx < BLK, j * BLK + lidx, N)
    midx_ref[0, :] = jnp.minimum(midx_ref[0, :], cand)


def _sc_gather_kernel(idx_hbm, d1_hbm, d2_hbm, o1_hbm, o2_hbm,
                      idx_v, rows_v, sem):
    base = lax.axis_index("s") * (B // 16)
    pltpu.sync_copy(idx_hbm.at[pl.ds(base, B // 16)], idx_v)
    pltpu.async_copy(d1_hbm.at[idx_v], rows_v, sem).wait()
    pltpu.sync_copy(rows_v, o1_hbm.at[pl.ds(base, B // 16)])
    pltpu.async_copy(d2_hbm.at[idx_v], rows_v, sem).wait()
    pltpu.sync_copy(rows_v, o2_hbm.at[pl.ds(base, B // 16)])


def _main_kernel(midx_ref, w1_ref, b1_ref, w2_ref, b2_ref, u1_ref, u2_ref,
                 d1f_ref, d1l_ref, d2f_ref, d2l_ref, slb_ref, out_ref,
                 x1_ref, x2_ref, u1c_ref, u2c_ref, oh1_ref, oh2_ref,
                 nx1_ref, nx2_ref, num1_ref, num2_ref):
    j = pl.program_id(0)
    sides = (
        (x1_ref, w1_ref, b1_ref, u1_ref, u1c_ref, oh1_ref, nx1_ref,
         d1f_ref, d1l_ref, num1_ref),
        (x2_ref, w2_ref, b2_ref, u2_ref, u2c_ref, oh2_ref, nx2_ref,
         d2f_ref, d2l_ref, num2_ref),
    )

    @pl.when(j == 0)
    def _init():
        ohq = (midx_ref[0, :][:, None]
               == jax.lax.broadcasted_iota(jnp.int32, (B, N), 1)
               ).astype(jnp.bfloat16)
        for (x_ref, w_ref, b_ref, u_ref, uc_ref, oh_ref, nx_ref,
             df_ref, _dl, num_ref) in sides:
            hi, mid, lo = _split3(df_ref[...])
            x_ref[...] = _dot(ohq, hi) + _dot(ohq, mid) + _dot(ohq, lo)
            u = u_ref[...]
            uc_ref[...] = _chunks8(_bits(u))
            xg = x_ref[...]
            y = jax.lax.dot_general(xg, w_ref[...], (((1,), (0,)), ((), ())),
                                    precision=HI,
                                    preferred_element_type=jnp.float32)
            y = y + b_ref[0, :][None, :]
            ny = jnp.sum(y * y, axis=1)
            nuf = jnp.sum(u * u, axis=1)
            dq = ny[:, None] + nuf[None, :] - 2.0 * _dot_t(y, u, HI)
            mn = jnp.min(dq, axis=1, keepdims=True)
            cb = jax.lax.broadcasted_iota(jnp.int32, (B, C), 1)
            cidx = jnp.min(jnp.where(dq == mn, cb, C), axis=1)
            oh_ref[...] = (cb == cidx[:, None]).astype(jnp.bfloat16)
            nx_ref[0, :] = jnp.sum(xg * xg, axis=1)
            num_ref[...] = jnp.zeros_like(num_ref)

    slb = slb_ref[...]
    slb_ext = jnp.concatenate(
        [slb, jnp.ones((BLK, 1), jnp.float32)], axis=1).astype(jnp.bfloat16)
    for (x_ref, _w, _b, _u, uc_ref, oh_ref, nx_ref,
         df_ref, dl_ref, num_ref) in sides:
        f = df_ref[pl.ds(pl.multiple_of(j * BLK, BLK), BLK), :]
        uc = uc_ref[...]
        lc = _chunks8(_bits(dl_ref[...]))
        nl = jnp.sum(lc.astype(jnp.float32) ** 2, axis=1)
        nu = jnp.sum(uc.astype(jnp.float32) ** 2, axis=1)
        m2l = nl[:, None] + nu[None, :] - 2.0 * _dot_t(lc, uc)
        e = (m2l == 0.0).astype(jnp.bfloat16)
        mask = _dot_t(oh_ref[...], e)
        g = _dot_t(x_ref[...].astype(jnp.bfloat16),
                   f.astype(jnp.bfloat16))
        nf = jnp.sum(f * f, axis=1)
        sq = nx_ref[0, :][:, None] + nf[None, :] - 2.0 * g
        expo = (jnp.exp(-sq) * mask).astype(jnp.bfloat16)
        num_ref[...] += jax.lax.dot_general(
            expo, slb_ext, (((1,), (0,)), ((), ())),
            preferred_element_type=jnp.float32)

    @pl.when(j == NB - 1)
    def _fin():
        n1 = num1_ref[...]
        n2 = num2_ref[...]
        out_ref[...] = 0.5 * (n1[:, :LD] / n1[:, LD:LD + 1]
                              + n2[:, :LD] / n2[:, LD:LD + 1])


def kernel(x, star_features, star_labels, d1_features, d1_labels,
           d2_features, d2_labels, unique1, unique2, W1, b1, W2, b2):
    f32 = jnp.float32
    midx = pl.pallas_call(
        _match_kernel,
        grid=(NB,),
        in_specs=[
            pl.BlockSpec((B, D), lambda j: (0, 0)),
            pl.BlockSpec((BLK, D), lambda j: (j, 0)),
        ],
        out_specs=pl.BlockSpec((1, B), lambda j: (0, 0)),
        out_shape=jax.ShapeDtypeStruct((1, B), jnp.int32),
        scratch_shapes=[
            pltpu.VMEM((B, CD), jnp.bfloat16),
        ],
    )(x, star_features)

    s = pl.pallas_call(
        _main_kernel,
        grid=(NB,),
        in_specs=[
            pl.BlockSpec((1, B), lambda j: (0, 0)),
            pl.BlockSpec((D, LD), lambda j: (0, 0)),
            pl.BlockSpec((1, LD), lambda j: (0, 0)),
            pl.BlockSpec((D, LD), lambda j: (0, 0)),
            pl.BlockSpec((1, LD), lambda j: (0, 0)),
            pl.BlockSpec((C, LD), lambda j: (0, 0)),
            pl.BlockSpec((C, LD), lambda j: (0, 0)),
            pl.BlockSpec((N, D), lambda j: (0, 0)),
            pl.BlockSpec((BLK, LD), lambda j: (j, 0)),
            pl.BlockSpec((N, D), lambda j: (0, 0)),
            pl.BlockSpec((BLK, LD), lambda j: (j, 0)),
            pl.BlockSpec((BLK, LD), lambda j: (j, 0)),
        ],
        out_specs=pl.BlockSpec((B, LD), lambda j: (0, 0)),
        out_shape=jax.ShapeDtypeStruct((B, LD), f32),
        scratch_shapes=[
            pltpu.VMEM((B, D), f32),
            pltpu.VMEM((B, D), f32),
            pltpu.VMEM((C, CLD), jnp.bfloat16),
            pltpu.VMEM((C, CLD), jnp.bfloat16),
            pltpu.VMEM((B, C), jnp.bfloat16),
            pltpu.VMEM((B, C), jnp.bfloat16),
            pltpu.VMEM((1, B), f32),
            pltpu.VMEM((1, B), f32),
            pltpu.VMEM((B, LD + 1), f32),
            pltpu.VMEM((B, LD + 1), f32),
        ],
    )(midx, W1, b1.reshape(1, LD), W2, b2.reshape(1, LD),
      unique1, unique2, d1_features, d1_labels, d2_features, d2_labels,
      star_labels)
    return s
